# Initial kernel scaffold; baseline (speedup 1.0000x reference)
import jax
import jax.numpy as jnp
from jax.experimental import pallas as pl


def kernel(pred, target):
    raise NotImplementedError("write your pallas kernel here")


# speedup vs baseline: 1198.2466x; 1198.2466x over previous
import jax
import jax.numpy as jnp
from jax import lax
from jax.experimental import pallas as pl
from jax.experimental.pallas import tpu as pltpu

_H = 224
_W = 224
_BIG = 3.0e4


def _shift(x, di, dj, fill, ii, jj):
    y = x
    if di:
        y = jnp.roll(y, -di, axis=0)
    if dj:
        y = jnp.roll(y, -dj, axis=1)
    ok = jnp.full(x.shape, True)
    if di:
        ok = ok & (ii + di >= 0) & (ii + di < _H)
    if dj:
        ok = ok & (jj + dj >= 0) & (jj + dj < _W)
    return jnp.where(ok, y, fill)


def _corner_mask(img, ii, jj):
    sh = lambda x, di, dj: _shift(x, di, dj, 0.0, ii, jj)
    ix = (-sh(img, -1, -1) + sh(img, -1, 1)
          - 2.0 * sh(img, 0, -1) + 2.0 * sh(img, 0, 1)
          - sh(img, 1, -1) + sh(img, 1, 1))
    iy = (-sh(img, -1, -1) - 2.0 * sh(img, -1, 0) - sh(img, -1, 1)
          + sh(img, 1, -1) + 2.0 * sh(img, 1, 0) + sh(img, 1, 1))
    ixx = ix * ix
    iyy = iy * iy
    ixy = ix * iy
    sxx = jnp.zeros_like(img)
    syy = jnp.zeros_like(img)
    sxy = jnp.zeros_like(img)
    for di in (-1, 0, 1):
        for dj in (-1, 0, 1):
            sxx = sxx + sh(ixx, di, dj)
            syy = syy + sh(iyy, di, dj)
            sxy = sxy + sh(ixy, di, dj)
    tr = sxx + syy
    disc = (sxx - syy) * (sxx - syy) + 4.0 * sxy * sxy
    eig = 0.5 * (tr - jnp.sqrt(jnp.maximum(disc, 0.0)))
    lmax = eig
    for di in range(-2, 3):
        for dj in range(-2, 3):
            if di == 0 and dj == 0:
                continue
            lmax = jnp.maximum(lmax, _shift(eig, di, dj, -jnp.inf, ii, jj))
    thresh = 0.01 * jnp.max(eig)
    mask = (eig >= lmax) & (eig > thresh)
    pe = jnp.where(mask, eig, 0.0)
    return (pe != 0.0).astype(jnp.float32)


def _edt_sq(cmask, g_ref, ii, jj, col_iota):
    d = jnp.where(cmask != 0.0, 0.0, _BIG)
    s = 1
    while s < _W:
        fs = float(s)
        d = jnp.minimum(d, _shift(d, 0, s, _BIG, ii, jj) + fs)
        d = jnp.minimum(d, _shift(d, 0, -s, _BIG, ii, jj) + fs)
        s *= 2
    g_ref[...] = d * d

    def body(r, acc):
        row = g_ref[pl.ds(r, 1), :]
        dr = col_iota - r.astype(jnp.float32)
        return jnp.minimum(acc, dr * dr + row)

    d0 = jnp.full((_H, _W), jnp.float32(4.0 * _BIG * _BIG))
    return lax.fori_loop(0, _H, body, d0)


def _body(pred_ref, tgt_ref, out_ref, g_ref):
    b = pl.program_id(0)
    ii = lax.broadcasted_iota(jnp.int32, (_H, _W), 0)
    jj = lax.broadcasted_iota(jnp.int32, (_H, _W), 1)
    col_iota = lax.broadcasted_iota(jnp.int32, (_H, 1), 0).astype(jnp.float32)

    img_p = (pred_ref[0, 0] > 0.0).astype(jnp.float32)
    img_t = tgt_ref[0, 0]

    cm_p = _corner_mask(img_p, ii, jj)
    cm_t = _corner_mask(img_t, ii, jj)

    dp = _edt_sq(cm_p, g_ref, ii, jj, col_iota)
    dt = _edt_sq(cm_t, g_ref, ii, jj, col_iota)

    np_cnt = jnp.sum(cm_p)
    ng_cnt = jnp.sum(cm_t)
    d1 = jnp.sqrt(jnp.maximum(dp, 1e-6))
    d2 = jnp.sqrt(jnp.maximum(dt, 1e-6))
    m1 = jnp.sum(jnp.where(cm_t != 0.0, d1, 0.0)) / jnp.maximum(ng_cnt, 1.0)
    m2 = jnp.sum(jnp.where(cm_p != 0.0, d2, 0.0)) / jnp.maximum(np_cnt, 1.0)
    dist = 0.5 * (m1 + m2)
    valid = (np_cnt > 0.0) & (ng_cnt > 0.0)
    part = jnp.where(valid, dist, 0.0)

    @pl.when(b == 0)
    def _():
        out_ref[...] = jnp.zeros((1, 1), jnp.float32)

    out_ref[...] += jnp.full((1, 1), part)


def kernel(pred, target):
    if pred.ndim == 3:
        pred = pred[:, None]
    if target.ndim == 3:
        target = target[:, None]
    bsz = pred.shape[0]
    out = pl.pallas_call(
        _body,
        grid=(bsz,),
        in_specs=[
            pl.BlockSpec((1, 1, _H, _W), lambda b: (b, 0, 0, 0)),
            pl.BlockSpec((1, 1, _H, _W), lambda b: (b, 0, 0, 0)),
        ],
        out_specs=pl.BlockSpec((1, 1), lambda b: (0, 0)),
        out_shape=jax.ShapeDtypeStruct((1, 1), jnp.float32),
        scratch_shapes=[pltpu.VMEM((_H, _W), jnp.float32)],
    )(pred.astype(jnp.float32), target.astype(jnp.float32))
    return out.reshape(1)
